# Initial kernel scaffold; baseline (speedup 1.0000x reference)
import jax
import jax.numpy as jnp
from jax.experimental import pallas as pl


def kernel(batch_mat, topk_edge, embedding, W, att_i, att_j, att_em_i, att_em_j, bias, gamma, beta):
    raise NotImplementedError("write your pallas kernel here")


# speedup vs baseline: 18.5207x; 18.5207x over previous
import jax
import jax.numpy as jnp
from jax import lax
from jax.experimental import pallas as pl
from jax.experimental.pallas import tpu as pltpu
from jax.experimental.pallas import tpu_sc as plsc

N, E, C = 10000, 320000, 128
D = 144
NC, NS, NW = 2, 16, 32
K = 80
EPW = E // NW
STEPS = EPW // K
NP = 10240
RPS = NP // NS
ROWB = 1000
LANES = 16


def _stage1_body(batch_ref, emb_ref, w_ref, ai_ref, aj_ref, aei_ref, aej_ref,
                 xaug_ref, aij_ref, bmax_ref):
    i = pl.program_id(0)
    x = lax.dot_general(batch_ref[...], w_ref[...], (((1,), (1,)), ((), ())),
                        preferred_element_type=jnp.float32)
    xaug_ref[:, :C] = x
    xaug_ref[:, C:C + 1] = jnp.ones((ROWB, 1), jnp.float32)
    xaug_ref[:, C + 1:] = jnp.zeros((ROWB, D - C - 1), jnp.float32)
    emb = emb_ref[...]
    ai = jnp.sum(x * ai_ref[...][None, :], axis=1) + \
        jnp.sum(emb * aei_ref[...][None, :], axis=1)
    aj = jnp.sum(x * aj_ref[...][None, :], axis=1) + \
        jnp.sum(emb * aej_ref[...][None, :], axis=1)
    aij_ref[...] = jnp.stack([ai, aj], axis=1)

    @pl.when(i == 0)
    def _():
        bmax_ref[...] = jnp.full((1, 2), -jnp.inf, jnp.float32)

    m = jnp.stack([jnp.max(ai), jnp.max(aj)])[None, :]
    bmax_ref[...] = jnp.maximum(bmax_ref[...], m)


_stage1 = pl.pallas_call(
    _stage1_body,
    grid=(N // ROWB,),
    in_specs=[
        pl.BlockSpec((ROWB, C), lambda i: (i, 0)),
        pl.BlockSpec((ROWB, C), lambda i: (i, 0)),
        pl.BlockSpec((C, C), lambda i: (0, 0)),
        pl.BlockSpec((C,), lambda i: (0,)),
        pl.BlockSpec((C,), lambda i: (0,)),
        pl.BlockSpec((C,), lambda i: (0,)),
        pl.BlockSpec((C,), lambda i: (0,)),
    ],
    out_specs=[
        pl.BlockSpec((ROWB, D), lambda i: (i, 0)),
        pl.BlockSpec((ROWB, 2), lambda i: (i, 0)),
        pl.BlockSpec((1, 2), lambda i: (0, 0)),
    ],
    out_shape=[
        jax.ShapeDtypeStruct((N, D), jnp.float32),
        jax.ShapeDtypeStruct((N, 2), jnp.float32),
        jax.ShapeDtypeStruct((1, 2), jnp.float32),
    ],
)


def _sc_body(xaug_hbm, aij_hbm, src_hbm, dst_hbm, bvec_hbm, acc_hbm,
             aij_t, sidx_t, didx_t, w_t, rows_t, b_t, acc_sh):
    cid = lax.axis_index("c")
    sid = lax.axis_index("s")
    wid = sid * NC + cid

    pltpu.sync_copy(aij_hbm, aij_t)
    pltpu.sync_copy(bvec_hbm, b_t)

    def _zrow(r, carry):
        for c in range(D // LANES):
            rows_t[r, pl.ds(c * LANES, LANES)] = \
                jnp.zeros((LANES,), jnp.float32)
        return carry

    lax.fori_loop(0, K, _zrow, 0)
    for t in range(RPS // K):
        pltpu.sync_copy(rows_t, acc_sh.at[pl.ds(sid * RPS + t * K, K)])
    plsc.subcore_barrier()

    o16 = jnp.ones((LANES,), jnp.int32)

    def _step(i, carry):
        base = wid * EPW + i * K
        pltpu.sync_copy(src_hbm.at[pl.ds(base, K)], sidx_t)
        pltpu.sync_copy(dst_hbm.at[pl.ds(base, K)], didx_t)
        bv = b_t[...]
        for v in range(K // LANES):
            s16 = sidx_t[pl.ds(v * LANES, LANES)]
            d16 = didx_t[pl.ds(v * LANES, LANES)]
            ai = plsc.load_gather(aij_t, (d16 * 2,))
            aj = plsc.load_gather(aij_t, (s16 * 2 + o16,))
            s = ai + aj
            al = jnp.where(s >= 0, s, 0.2 * s)
            w = jnp.exp(al - bv)
            w = jnp.where(s16 == d16, jnp.float32(0.0), w)
            w_t[pl.ds(v * LANES, LANES)] = w
        pltpu.sync_copy(xaug_hbm.at[sidx_t], rows_t)

        def _scale(j, c2):
            wj = plsc.load_gather(w_t, (jnp.broadcast_to(j, (LANES,)),))
            for c in range(D // LANES):
                rows_t[j, pl.ds(c * LANES, LANES)] = \
                    rows_t[j, pl.ds(c * LANES, LANES)] * wj
            return c2

        lax.fori_loop(0, K, _scale, 0)
        pltpu.sync_copy(rows_t, acc_sh.at[didx_t], add=True)
        return carry

    lax.fori_loop(0, STEPS, _step, 0)
    plsc.subcore_barrier()
    pltpu.sync_copy(acc_sh.at[pl.ds(sid * RPS, RPS)],
                    acc_hbm.at[cid, pl.ds(sid * RPS, RPS)])


_sc_edges = pl.kernel(
    _sc_body,
    out_type=jax.ShapeDtypeStruct((NC, NP, D), jnp.float32),
    mesh=plsc.VectorSubcoreMesh(core_axis_name="c", subcore_axis_name="s"),
    compiler_params=pltpu.CompilerParams(needs_layout_passes=False, use_tc_tiling_on_sc=False),
    scratch_types=[
        pltpu.VMEM((2 * N,), jnp.float32),
        pltpu.VMEM((K,), jnp.int32),
        pltpu.VMEM((K,), jnp.int32),
        pltpu.VMEM((K,), jnp.float32),
        pltpu.VMEM((K, D), jnp.float32),
        pltpu.VMEM((LANES,), jnp.float32),
        pltpu.VMEM_SHARED((NP, D), jnp.float32),
    ],
)


def _stage3_body(acc_ref, xaug_ref, aij_ref, bs_ref, bias_ref, gamma_ref,
                 beta_ref, out_ref):
    B = bs_ref[0, 0]
    x = xaug_ref[:, :C]
    s = aij_ref[:, 0:1] + aij_ref[:, 1:2]
    al = jnp.where(s >= 0, s, 0.2 * s)
    exs = jnp.exp(al - B)
    num = acc_ref[0, :N, :C] + acc_ref[1, :N, :C] + exs * x
    den = acc_ref[0, :N, C:C + 1] + acc_ref[1, :N, C:C + 1] + exs
    o = num / jnp.maximum(den, 1e-16) + bias_ref[...][None, :]
    mean = jnp.mean(o, axis=0, keepdims=True)
    var = jnp.mean((o - mean) ** 2, axis=0, keepdims=True)
    o = (o - mean) / jnp.sqrt(var + 1e-5) * gamma_ref[...][None, :] + \
        beta_ref[...][None, :]
    out_ref[...] = jnp.maximum(o, 0.0)


_stage3 = pl.pallas_call(
    _stage3_body,
    out_shape=jax.ShapeDtypeStruct((N, C), jnp.float32),
)


def kernel(batch_mat, topk_edge, embedding, W, att_i, att_j, att_em_i,
           att_em_j, bias, gamma, beta):
    xaug, aij, bmax = _stage1(batch_mat, embedding, W, att_i, att_j,
                              att_em_i, att_em_j)
    ssum = bmax[0, 0] + bmax[0, 1]
    B = jnp.where(ssum >= 0, ssum, 0.2 * ssum)
    acc = _sc_edges(xaug, aij.reshape(2 * N), topk_edge[0], topk_edge[1],
                    jnp.broadcast_to(B, (LANES,)))
    return _stage3(acc, xaug, aij, B.reshape(1, 1), bias, gamma, beta)


# speedup vs baseline: 27.3568x; 1.4771x over previous
import jax
import jax.numpy as jnp
from jax import lax
from jax.experimental import pallas as pl
from jax.experimental.pallas import tpu as pltpu
from jax.experimental.pallas import tpu_sc as plsc

N, E, C = 10000, 320000, 128
D = 136
NC, NS, NW = 2, 16, 32
K = 80
EPW = E // NW
STEPS = EPW // K
NP = 10240
RPS = NP // NS
ROWB = 1000
LANES = 16


def _stage1_body(batch_ref, emb_ref, w_ref, ai_ref, aj_ref, aei_ref, aej_ref,
                 xaug_ref, aij_ref, bmax_ref):
    i = pl.program_id(0)
    x = lax.dot_general(batch_ref[...], w_ref[...], (((1,), (1,)), ((), ())),
                        preferred_element_type=jnp.float32)
    xaug_ref[:, :C] = x
    xaug_ref[:, C:C + 1] = jnp.ones((ROWB, 1), jnp.float32)
    xaug_ref[:, C + 1:] = jnp.zeros((ROWB, D - C - 1), jnp.float32)
    emb = emb_ref[...]
    ai = jnp.sum(x * ai_ref[...][None, :], axis=1) + \
        jnp.sum(emb * aei_ref[...][None, :], axis=1)
    aj = jnp.sum(x * aj_ref[...][None, :], axis=1) + \
        jnp.sum(emb * aej_ref[...][None, :], axis=1)
    aij_ref[...] = jnp.stack([ai, aj], axis=1)

    @pl.when(i == 0)
    def _():
        bmax_ref[...] = jnp.full((1, 2), -jnp.inf, jnp.float32)

    m = jnp.stack([jnp.max(ai), jnp.max(aj)])[None, :]
    bmax_ref[...] = jnp.maximum(bmax_ref[...], m)


_stage1 = pl.pallas_call(
    _stage1_body,
    grid=(N // ROWB,),
    in_specs=[
        pl.BlockSpec((ROWB, C), lambda i: (i, 0)),
        pl.BlockSpec((ROWB, C), lambda i: (i, 0)),
        pl.BlockSpec((C, C), lambda i: (0, 0)),
        pl.BlockSpec((C,), lambda i: (0,)),
        pl.BlockSpec((C,), lambda i: (0,)),
        pl.BlockSpec((C,), lambda i: (0,)),
        pl.BlockSpec((C,), lambda i: (0,)),
    ],
    out_specs=[
        pl.BlockSpec((ROWB, D), lambda i: (i, 0)),
        pl.BlockSpec((ROWB, 2), lambda i: (i, 0)),
        pl.BlockSpec((1, 2), lambda i: (0, 0)),
    ],
    out_shape=[
        jax.ShapeDtypeStruct((N, D), jnp.float32),
        jax.ShapeDtypeStruct((N, 2), jnp.float32),
        jax.ShapeDtypeStruct((1, 2), jnp.float32),
    ],
)


def _lane_bcast(vec, lane):
    return lax.gather(
        vec, jnp.full((LANES, 1), lane, jnp.int32),
        lax.GatherDimensionNumbers(offset_dims=(), collapsed_slice_dims=(0,),
                                   start_index_map=(0,)),
        (1,), mode=lax.GatherScatterMode.PROMISE_IN_BOUNDS)


def _sc_body(xaug_hbm, aij_hbm, src_hbm, dst_hbm, bvec_hbm, acc_hbm,
             aij_t, sidx0, didx0, rows0, sidx1, didx1, rows1, w_t, b_t,
             acc_sh, sem0, sem1):
    cid = lax.axis_index("c")
    sid = lax.axis_index("s")
    wid = sid * NC + cid
    bufs = ((sidx0, didx0, rows0, sem0), (sidx1, didx1, rows1, sem1))

    pltpu.sync_copy(aij_hbm, aij_t)
    pltpu.sync_copy(bvec_hbm, b_t)

    def _zrow(r, carry):
        for c in range(C // LANES):
            rows0[r, pl.ds(c * LANES, LANES)] = \
                jnp.zeros((LANES,), jnp.float32)
        rows0[r, pl.ds(D - LANES, LANES)] = jnp.zeros((LANES,), jnp.float32)
        return carry

    lax.fori_loop(0, K, _zrow, 0)
    for t in range(RPS // K):
        pltpu.sync_copy(rows0, acc_sh.at[pl.ds(sid * RPS + t * K, K)])
    plsc.subcore_barrier()

    o16 = jnp.ones((LANES,), jnp.int32)
    ebase = wid * EPW

    def _load_issue(buf, base):
        sidx, didx, rows, sem = buf
        pltpu.sync_copy(src_hbm.at[pl.ds(base, K)], sidx)
        pltpu.sync_copy(dst_hbm.at[pl.ds(base, K)], didx)
        pltpu.async_copy(xaug_hbm.at[sidx], rows, sem)

    def _process(buf):
        sidx, didx, rows, sem = buf
        bv = b_t[...]
        for v in range(K // LANES):
            s16 = sidx[pl.ds(v * LANES, LANES)]
            d16 = didx[pl.ds(v * LANES, LANES)]
            ai = plsc.load_gather(aij_t, (d16 * 2,))
            aj = plsc.load_gather(aij_t, (s16 * 2 + o16,))
            s = ai + aj
            al = jnp.where(s >= 0, s, 0.2 * s)
            w = jnp.exp(al - bv)
            w = jnp.where(s16 == d16, jnp.float32(0.0), w)
            w_t[pl.ds(v * LANES, LANES)] = w
        pltpu.make_async_copy(xaug_hbm.at[sidx], rows, sem).wait()

        def _scale(v, c2):
            w16 = w_t[pl.ds(v * LANES, LANES)]
            for lane in range(LANES):
                wl = _lane_bcast(w16, lane)
                j = v * LANES + lane
                tail = rows[j, pl.ds(D - LANES, LANES)]
                for c in range(C // LANES):
                    rows[j, pl.ds(c * LANES, LANES)] = \
                        rows[j, pl.ds(c * LANES, LANES)] * wl
                rows[j, pl.ds(D - LANES, LANES)] = tail * wl
            return c2

        lax.fori_loop(0, K // LANES, _scale, 0)
        pltpu.sync_copy(rows, acc_sh.at[didx], add=True)

    _load_issue(bufs[0], ebase)

    def _outer(g, carry):
        for b in range(2):
            i2 = g * 2 + b
            _load_issue(bufs[(b + 1) % 2], ebase + (i2 + 1) * K)
            _process(bufs[b])
        return carry

    lax.fori_loop(0, (STEPS - 1) // 2, _outer, 0)
    _process(bufs[(STEPS - 1) % 2])

    plsc.subcore_barrier()
    pltpu.sync_copy(acc_sh.at[pl.ds(sid * RPS, RPS)],
                    acc_hbm.at[cid, pl.ds(sid * RPS, RPS)])


_sc_edges = pl.kernel(
    _sc_body,
    out_type=jax.ShapeDtypeStruct((NC, NP, D), jnp.float32),
    mesh=plsc.VectorSubcoreMesh(core_axis_name="c", subcore_axis_name="s"),
    compiler_params=pltpu.CompilerParams(needs_layout_passes=False, use_tc_tiling_on_sc=False),
    scratch_types=[
        pltpu.VMEM((2 * N,), jnp.float32),
        pltpu.VMEM((K,), jnp.int32),
        pltpu.VMEM((K,), jnp.int32),
        pltpu.VMEM((K, D), jnp.float32),
        pltpu.VMEM((K,), jnp.int32),
        pltpu.VMEM((K,), jnp.int32),
        pltpu.VMEM((K, D), jnp.float32),
        pltpu.VMEM((K,), jnp.float32),
        pltpu.VMEM((LANES,), jnp.float32),
        pltpu.VMEM_SHARED((NP, D), jnp.float32),
        pltpu.SemaphoreType.DMA,
        pltpu.SemaphoreType.DMA,
    ],
)


def _stage3_body(acc_ref, xaug_ref, aij_ref, bs_ref, bias_ref, gamma_ref,
                 beta_ref, out_ref):
    B = bs_ref[0, 0]
    x = xaug_ref[:, :C]
    s = aij_ref[:, 0:1] + aij_ref[:, 1:2]
    al = jnp.where(s >= 0, s, 0.2 * s)
    exs = jnp.exp(al - B)
    num = acc_ref[0, :N, :C] + acc_ref[1, :N, :C] + exs * x
    den = acc_ref[0, :N, C:C + 1] + acc_ref[1, :N, C:C + 1] + exs
    o = num / jnp.maximum(den, 1e-16) + bias_ref[...][None, :]
    mean = jnp.mean(o, axis=0, keepdims=True)
    var = jnp.mean((o - mean) ** 2, axis=0, keepdims=True)
    o = (o - mean) / jnp.sqrt(var + 1e-5) * gamma_ref[...][None, :] + \
        beta_ref[...][None, :]
    out_ref[...] = jnp.maximum(o, 0.0)


_stage3 = pl.pallas_call(
    _stage3_body,
    out_shape=jax.ShapeDtypeStruct((N, C), jnp.float32),
)


def kernel(batch_mat, topk_edge, embedding, W, att_i, att_j, att_em_i,
           att_em_j, bias, gamma, beta):
    xaug, aij, bmax = _stage1(batch_mat, embedding, W, att_i, att_j,
                              att_em_i, att_em_j)
    ssum = bmax[0, 0] + bmax[0, 1]
    B = jnp.where(ssum >= 0, ssum, 0.2 * ssum)
    acc = _sc_edges(xaug, aij.reshape(2 * N), topk_edge[0], topk_edge[1],
                    jnp.broadcast_to(B, (LANES,)))
    return _stage3(acc, xaug, aij, B.reshape(1, 1), bias, gamma, beta)


# speedup vs baseline: 30.1091x; 1.1006x over previous
import jax
import jax.numpy as jnp
from jax import lax
from jax.experimental import pallas as pl
from jax.experimental.pallas import tpu as pltpu
from jax.experimental.pallas import tpu_sc as plsc

N, E, C = 10000, 320000, 128
D = 136
NC, NS, NW = 2, 16, 32
K = 80
EPW = E // NW
STEPS = EPW // K
NP = 10240
RPS = NP // NS
ROWB = 1000
LANES = 16


def _stage1_body(batch_ref, emb_ref, w_ref, ai_ref, aj_ref, aei_ref, aej_ref,
                 xaug_ref, aij_ref, bmax_ref):
    i = pl.program_id(0)
    x = lax.dot_general(batch_ref[...], w_ref[...], (((1,), (1,)), ((), ())),
                        preferred_element_type=jnp.float32)
    xaug_ref[:, :C] = x
    xaug_ref[:, C:C + 1] = jnp.ones((ROWB, 1), jnp.float32)
    xaug_ref[:, C + 1:] = jnp.zeros((ROWB, D - C - 1), jnp.float32)
    emb = emb_ref[...]
    ai = jnp.sum(x * ai_ref[...][None, :], axis=1) + \
        jnp.sum(emb * aei_ref[...][None, :], axis=1)
    aj = jnp.sum(x * aj_ref[...][None, :], axis=1) + \
        jnp.sum(emb * aej_ref[...][None, :], axis=1)
    aij_ref[...] = jnp.stack([ai, aj], axis=1)

    @pl.when(i == 0)
    def _():
        bmax_ref[...] = jnp.full((1, 2), -jnp.inf, jnp.float32)

    m = jnp.stack([jnp.max(ai), jnp.max(aj)])[None, :]
    bmax_ref[...] = jnp.maximum(bmax_ref[...], m)


_stage1 = pl.pallas_call(
    _stage1_body,
    grid=(N // ROWB,),
    in_specs=[
        pl.BlockSpec((ROWB, C), lambda i: (i, 0)),
        pl.BlockSpec((ROWB, C), lambda i: (i, 0)),
        pl.BlockSpec((C, C), lambda i: (0, 0)),
        pl.BlockSpec((C,), lambda i: (0,)),
        pl.BlockSpec((C,), lambda i: (0,)),
        pl.BlockSpec((C,), lambda i: (0,)),
        pl.BlockSpec((C,), lambda i: (0,)),
    ],
    out_specs=[
        pl.BlockSpec((ROWB, D), lambda i: (i, 0)),
        pl.BlockSpec((ROWB, 2), lambda i: (i, 0)),
        pl.BlockSpec((1, 2), lambda i: (0, 0)),
    ],
    out_shape=[
        jax.ShapeDtypeStruct((N, D), jnp.float32),
        jax.ShapeDtypeStruct((N, 2), jnp.float32),
        jax.ShapeDtypeStruct((1, 2), jnp.float32),
    ],
)


def _lane_bcast(vec, lane):
    return lax.gather(
        vec, jnp.full((LANES, 1), lane, jnp.int32),
        lax.GatherDimensionNumbers(offset_dims=(), collapsed_slice_dims=(0,),
                                   start_index_map=(0,)),
        (1,), mode=lax.GatherScatterMode.PROMISE_IN_BOUNDS)


def _sc_body(xaug_hbm, aij_hbm, src_hbm, dst_hbm, bvec_hbm, acc_hbm,
             aij_t, sidx0, didx0, rows0, sidx1, didx1, rows1, w_t, b_t,
             acc_sh, sg0, sg1, ss0, ss1, si0, si1):
    cid = lax.axis_index("c")
    sid = lax.axis_index("s")
    wid = sid * NC + cid
    bufs = ((sidx0, didx0, rows0, sg0, ss0, si0),
            (sidx1, didx1, rows1, sg1, ss1, si1))

    pltpu.sync_copy(aij_hbm, aij_t)
    pltpu.sync_copy(bvec_hbm, b_t)

    def _zrow(r, carry):
        for c in range(C // LANES):
            rows0[r, pl.ds(c * LANES, LANES)] = \
                jnp.zeros((LANES,), jnp.float32)
        rows0[r, pl.ds(D - LANES, LANES)] = jnp.zeros((LANES,), jnp.float32)
        return carry

    lax.fori_loop(0, K, _zrow, 0)
    for t in range(RPS // K):
        pltpu.sync_copy(rows0, acc_sh.at[pl.ds(sid * RPS + t * K, K)])
    plsc.subcore_barrier()

    o16 = jnp.ones((LANES,), jnp.int32)
    ebase = wid * EPW

    def _weights(sidx, didx):
        bv = b_t[...]
        for v in range(K // LANES):
            s16 = sidx[pl.ds(v * LANES, LANES)]
            d16 = didx[pl.ds(v * LANES, LANES)]
            ai = plsc.load_gather(aij_t, (d16 * 2,))
            aj = plsc.load_gather(aij_t, (s16 * 2 + o16,))
            s = ai + aj
            al = jnp.where(s >= 0, s, 0.2 * s)
            w = jnp.exp(al - bv)
            w = jnp.where(s16 == d16, jnp.float32(0.0), w)
            w_t[pl.ds(v * LANES, LANES)] = w

    def _scale_rows(rows):
        def _scale(v, c2):
            w16 = w_t[pl.ds(v * LANES, LANES)]
            for lane in range(LANES):
                wl = _lane_bcast(w16, lane)
                j = v * LANES + lane
                tail = rows[j, pl.ds(D - LANES, LANES)]
                for c in range(C // LANES):
                    rows[j, pl.ds(c * LANES, LANES)] = \
                        rows[j, pl.ds(c * LANES, LANES)] * wl
                rows[j, pl.ds(D - LANES, LANES)] = tail * wl
            return c2

        lax.fori_loop(0, K // LANES, _scale, 0)

    sidxP, didxP, rowsP, sgP, ssP, siP = bufs[0]
    pltpu.sync_copy(src_hbm.at[pl.ds(ebase, K)], sidxP)
    pltpu.sync_copy(dst_hbm.at[pl.ds(ebase, K)], didxP)
    pltpu.async_copy(xaug_hbm.at[sidxP], rowsP, sgP)

    def _iter(i2, b):
        sidx, didx, rows, sg, ss, si = bufs[b]
        osidx, odidx, orows, osg, oss, osi = bufs[1 - b]
        nbase = ebase + (i2 + 1) * K

        if b == 0:
            @pl.when(i2 > 0)
            def _():
                pltpu.make_async_copy(orows, acc_sh.at[odidx], oss).wait()
        else:
            pltpu.make_async_copy(orows, acc_sh.at[odidx], oss).wait()
        pltpu.async_copy(src_hbm.at[pl.ds(nbase, K)], osidx, osi)
        pltpu.async_copy(dst_hbm.at[pl.ds(nbase, K)], odidx, osi)
        _weights(sidx, didx)
        pltpu.make_async_copy(xaug_hbm.at[sidx], rows, sg).wait()
        _scale_rows(rows)
        pltpu.make_async_copy(src_hbm.at[pl.ds(nbase, K)], osidx, osi).wait()
        pltpu.make_async_copy(dst_hbm.at[pl.ds(nbase, K)], odidx, osi).wait()
        pltpu.async_copy(xaug_hbm.at[osidx], orows, osg)
        pltpu.async_copy(rows, acc_sh.at[didx], ss, add=True)

    def _outer(g, carry):
        for b in range(2):
            _iter(g * 2 + b, b)
        return carry

    lax.fori_loop(0, (STEPS - 1) // 2, _outer, 0)

    sidxE, didxE, rowsE, sgE, ssE, siE = bufs[0]
    pltpu.make_async_copy(rows1, acc_sh.at[didx1], ss1).wait()
    _weights(sidxE, didxE)
    pltpu.make_async_copy(xaug_hbm.at[sidxE], rowsE, sgE).wait()
    _scale_rows(rowsE)
    pltpu.sync_copy(rowsE, acc_sh.at[didxE], add=True)

    plsc.subcore_barrier()
    pltpu.sync_copy(acc_sh.at[pl.ds(sid * RPS, RPS)],
                    acc_hbm.at[cid, pl.ds(sid * RPS, RPS)])


_sc_edges = pl.kernel(
    _sc_body,
    out_type=jax.ShapeDtypeStruct((NC, NP, D), jnp.float32),
    mesh=plsc.VectorSubcoreMesh(core_axis_name="c", subcore_axis_name="s"),
    compiler_params=pltpu.CompilerParams(needs_layout_passes=False, use_tc_tiling_on_sc=False),
    scratch_types=[
        pltpu.VMEM((2 * N,), jnp.float32),
        pltpu.VMEM((K,), jnp.int32),
        pltpu.VMEM((K,), jnp.int32),
        pltpu.VMEM((K, D), jnp.float32),
        pltpu.VMEM((K,), jnp.int32),
        pltpu.VMEM((K,), jnp.int32),
        pltpu.VMEM((K, D), jnp.float32),
        pltpu.VMEM((K,), jnp.float32),
        pltpu.VMEM((LANES,), jnp.float32),
        pltpu.VMEM_SHARED((NP, D), jnp.float32),
        pltpu.SemaphoreType.DMA,
        pltpu.SemaphoreType.DMA,
        pltpu.SemaphoreType.DMA,
        pltpu.SemaphoreType.DMA,
        pltpu.SemaphoreType.DMA,
        pltpu.SemaphoreType.DMA,
    ],
)


def _stage3_body(acc_ref, xaug_ref, aij_ref, bs_ref, bias_ref, gamma_ref,
                 beta_ref, out_ref):
    B = bs_ref[0, 0]
    x = xaug_ref[:, :C]
    s = aij_ref[:, 0:1] + aij_ref[:, 1:2]
    al = jnp.where(s >= 0, s, 0.2 * s)
    exs = jnp.exp(al - B)
    num = acc_ref[0, :N, :C] + acc_ref[1, :N, :C] + exs * x
    den = acc_ref[0, :N, C:C + 1] + acc_ref[1, :N, C:C + 1] + exs
    o = num / jnp.maximum(den, 1e-16) + bias_ref[...][None, :]
    mean = jnp.mean(o, axis=0, keepdims=True)
    var = jnp.mean((o - mean) ** 2, axis=0, keepdims=True)
    o = (o - mean) / jnp.sqrt(var + 1e-5) * gamma_ref[...][None, :] + \
        beta_ref[...][None, :]
    out_ref[...] = jnp.maximum(o, 0.0)


_stage3 = pl.pallas_call(
    _stage3_body,
    out_shape=jax.ShapeDtypeStruct((N, C), jnp.float32),
)


def kernel(batch_mat, topk_edge, embedding, W, att_i, att_j, att_em_i,
           att_em_j, bias, gamma, beta):
    xaug, aij, bmax = _stage1(batch_mat, embedding, W, att_i, att_j,
                              att_em_i, att_em_j)
    ssum = bmax[0, 0] + bmax[0, 1]
    B = jnp.where(ssum >= 0, ssum, 0.2 * ssum)
    acc = _sc_edges(xaug, aij.reshape(2 * N), topk_edge[0], topk_edge[1],
                    jnp.broadcast_to(B, (LANES,)))
    return _stage3(acc, xaug, aij, B.reshape(1, 1), bias, gamma, beta)


# speedup vs baseline: 30.1496x; 1.0013x over previous
import jax
import jax.numpy as jnp
from jax import lax
from jax.experimental import pallas as pl
from jax.experimental.pallas import tpu as pltpu
from jax.experimental.pallas import tpu_sc as plsc

N, E, C = 10000, 320000, 128
D = 136
NC, NS, NW = 2, 16, 32
K = 80
EPW = E // NW
STEPS = EPW // K
NP = 10240
RPS = NP // NS
ROWB = 1000
LANES = 16


def _stage1_body(batch_ref, emb_ref, w_ref, ai_ref, aj_ref, aei_ref, aej_ref,
                 xaug_ref, aij_ref, bmax_ref):
    i = pl.program_id(0)
    x = lax.dot_general(batch_ref[...], w_ref[...], (((1,), (1,)), ((), ())),
                        preferred_element_type=jnp.float32)
    xaug_ref[:, :C] = x
    xaug_ref[:, C:C + 1] = jnp.ones((ROWB, 1), jnp.float32)
    xaug_ref[:, C + 1:] = jnp.zeros((ROWB, D - C - 1), jnp.float32)
    emb = emb_ref[...]
    ai = jnp.sum(x * ai_ref[...][None, :], axis=1) + \
        jnp.sum(emb * aei_ref[...][None, :], axis=1)
    aj = jnp.sum(x * aj_ref[...][None, :], axis=1) + \
        jnp.sum(emb * aej_ref[...][None, :], axis=1)
    aij_ref[...] = jnp.stack([ai, aj], axis=1)

    @pl.when(i == 0)
    def _():
        bmax_ref[...] = jnp.full((1, 2), -jnp.inf, jnp.float32)

    m = jnp.stack([jnp.max(ai), jnp.max(aj)])[None, :]
    bmax_ref[...] = jnp.maximum(bmax_ref[...], m)


_stage1 = pl.pallas_call(
    _stage1_body,
    grid=(N // ROWB,),
    in_specs=[
        pl.BlockSpec((ROWB, C), lambda i: (i, 0)),
        pl.BlockSpec((ROWB, C), lambda i: (i, 0)),
        pl.BlockSpec((C, C), lambda i: (0, 0)),
        pl.BlockSpec((C,), lambda i: (0,)),
        pl.BlockSpec((C,), lambda i: (0,)),
        pl.BlockSpec((C,), lambda i: (0,)),
        pl.BlockSpec((C,), lambda i: (0,)),
    ],
    out_specs=[
        pl.BlockSpec((ROWB, D), lambda i: (i, 0)),
        pl.BlockSpec((ROWB, 2), lambda i: (i, 0)),
        pl.BlockSpec((1, 2), lambda i: (0, 0)),
    ],
    out_shape=[
        jax.ShapeDtypeStruct((N, D), jnp.float32),
        jax.ShapeDtypeStruct((N, 2), jnp.float32),
        jax.ShapeDtypeStruct((1, 2), jnp.float32),
    ],
)


def _lane_bcast(vec, lane):
    return lax.gather(
        vec, jnp.full((LANES, 1), lane, jnp.int32),
        lax.GatherDimensionNumbers(offset_dims=(), collapsed_slice_dims=(0,),
                                   start_index_map=(0,)),
        (1,), mode=lax.GatherScatterMode.PROMISE_IN_BOUNDS)


def _sc_body(xaug_hbm, aij_hbm, src_hbm, dst_hbm, bvec_hbm, acc_hbm,
             aij_t, sidx0, didx0, rows0, sidx1, didx1, rows1, w_t, b_t,
             acc_sh, sg0, sg1, ss0, ss1, si0, si1):
    cid = lax.axis_index("c")
    sid = lax.axis_index("s")
    wid = sid * NC + cid
    bufs = ((sidx0, didx0, rows0, sg0, ss0, si0),
            (sidx1, didx1, rows1, sg1, ss1, si1))

    pltpu.sync_copy(aij_hbm, aij_t)
    pltpu.sync_copy(bvec_hbm, b_t)

    def _zrow(r, carry):
        for c in range(C // LANES):
            rows0[r, pl.ds(c * LANES, LANES)] = \
                jnp.zeros((LANES,), jnp.float32)
        rows0[r, pl.ds(D - LANES, LANES)] = jnp.zeros((LANES,), jnp.float32)
        return carry

    lax.fori_loop(0, K, _zrow, 0)
    for t in range(RPS // K):
        pltpu.sync_copy(rows0, acc_sh.at[pl.ds(sid * RPS + t * K, K)])
    plsc.subcore_barrier()

    o16 = jnp.ones((LANES,), jnp.int32)
    ebase = wid * EPW

    def _weights(sidx, didx):
        bv = b_t[...]
        for v in range(K // LANES):
            s16 = sidx[pl.ds(v * LANES, LANES)]
            d16 = didx[pl.ds(v * LANES, LANES)]
            ai = plsc.load_gather(aij_t, (d16 * 2,))
            aj = plsc.load_gather(aij_t, (s16 * 2 + o16,))
            s = ai + aj
            al = jnp.where(s >= 0, s, 0.2 * s)
            w = jnp.exp(al - bv)
            w = jnp.where(s16 == d16, jnp.float32(0.0), w)
            w_t[pl.ds(v * LANES, LANES)] = w

    def _scale_rows(rows):
        def _scale(v, c2):
            w16 = w_t[pl.ds(v * LANES, LANES)]
            for lane in range(LANES):
                wl = _lane_bcast(w16, lane)
                j = v * LANES + lane
                tail = rows[j, pl.ds(D - LANES, LANES)]
                for c in range(C // LANES):
                    rows[j, pl.ds(c * LANES, LANES)] = \
                        rows[j, pl.ds(c * LANES, LANES)] * wl
                rows[j, pl.ds(D - LANES, LANES)] = tail * wl
            return c2

        lax.fori_loop(0, K // LANES, _scale, 0)

    sidxP, didxP, rowsP, sgP, ssP, siP = bufs[0]
    pltpu.sync_copy(src_hbm.at[pl.ds(ebase, K)], sidxP)
    pltpu.sync_copy(dst_hbm.at[pl.ds(ebase, K)], didxP)
    pltpu.async_copy(xaug_hbm.at[sidxP], rowsP, sgP)

    def _iter(i2, b):
        sidx, didx, rows, sg, ss, si = bufs[b]
        osidx, odidx, orows, osg, oss, osi = bufs[1 - b]
        nbase = ebase + (i2 + 1) * K

        if b == 0:
            @pl.when(i2 > 0)
            def _():
                pltpu.make_async_copy(orows, acc_sh.at[odidx], oss).wait()
        else:
            pltpu.make_async_copy(orows, acc_sh.at[odidx], oss).wait()
        pltpu.async_copy(src_hbm.at[pl.ds(nbase, K)], osidx, osi)
        pltpu.async_copy(dst_hbm.at[pl.ds(nbase, K)], odidx, osi)
        _weights(sidx, didx)
        pltpu.make_async_copy(xaug_hbm.at[sidx], rows, sg).wait()
        _scale_rows(rows)
        pltpu.make_async_copy(src_hbm.at[pl.ds(nbase, K)], osidx, osi).wait()
        pltpu.make_async_copy(dst_hbm.at[pl.ds(nbase, K)], odidx, osi).wait()
        pltpu.async_copy(xaug_hbm.at[osidx], orows, osg)
        pltpu.async_copy(rows, acc_sh.at[didx], ss, priority=1, add=True)

    def _outer(g, carry):
        for b in range(2):
            _iter(g * 2 + b, b)
        return carry

    lax.fori_loop(0, (STEPS - 1) // 2, _outer, 0)

    sidxE, didxE, rowsE, sgE, ssE, siE = bufs[0]
    pltpu.make_async_copy(rows1, acc_sh.at[didx1], ss1).wait()
    _weights(sidxE, didxE)
    pltpu.make_async_copy(xaug_hbm.at[sidxE], rowsE, sgE).wait()
    _scale_rows(rowsE)
    pltpu.sync_copy(rowsE, acc_sh.at[didxE], add=True)

    plsc.subcore_barrier()
    pltpu.sync_copy(acc_sh.at[pl.ds(sid * RPS, RPS)],
                    acc_hbm.at[cid, pl.ds(sid * RPS, RPS)])


_sc_edges = pl.kernel(
    _sc_body,
    out_type=jax.ShapeDtypeStruct((NC, NP, D), jnp.float32),
    mesh=plsc.VectorSubcoreMesh(core_axis_name="c", subcore_axis_name="s"),
    compiler_params=pltpu.CompilerParams(needs_layout_passes=False, use_tc_tiling_on_sc=False),
    scratch_types=[
        pltpu.VMEM((2 * N,), jnp.float32),
        pltpu.VMEM((K,), jnp.int32),
        pltpu.VMEM((K,), jnp.int32),
        pltpu.VMEM((K, D), jnp.float32),
        pltpu.VMEM((K,), jnp.int32),
        pltpu.VMEM((K,), jnp.int32),
        pltpu.VMEM((K, D), jnp.float32),
        pltpu.VMEM((K,), jnp.float32),
        pltpu.VMEM((LANES,), jnp.float32),
        pltpu.VMEM_SHARED((NP, D), jnp.float32),
        pltpu.SemaphoreType.DMA,
        pltpu.SemaphoreType.DMA,
        pltpu.SemaphoreType.DMA,
        pltpu.SemaphoreType.DMA,
        pltpu.SemaphoreType.DMA,
        pltpu.SemaphoreType.DMA,
    ],
)


def _stage3_body(acc_ref, xaug_ref, aij_ref, bs_ref, bias_ref, gamma_ref,
                 beta_ref, out_ref):
    B = bs_ref[0, 0]
    x = xaug_ref[:, :C]
    s = aij_ref[:, 0:1] + aij_ref[:, 1:2]
    al = jnp.where(s >= 0, s, 0.2 * s)
    exs = jnp.exp(al - B)
    num = acc_ref[0, :N, :C] + acc_ref[1, :N, :C] + exs * x
    den = acc_ref[0, :N, C:C + 1] + acc_ref[1, :N, C:C + 1] + exs
    o = num / jnp.maximum(den, 1e-16) + bias_ref[...][None, :]
    mean = jnp.mean(o, axis=0, keepdims=True)
    var = jnp.mean((o - mean) ** 2, axis=0, keepdims=True)
    o = (o - mean) / jnp.sqrt(var + 1e-5) * gamma_ref[...][None, :] + \
        beta_ref[...][None, :]
    out_ref[...] = jnp.maximum(o, 0.0)


_stage3 = pl.pallas_call(
    _stage3_body,
    out_shape=jax.ShapeDtypeStruct((N, C), jnp.float32),
)


def kernel(batch_mat, topk_edge, embedding, W, att_i, att_j, att_em_i,
           att_em_j, bias, gamma, beta):
    xaug, aij, bmax = _stage1(batch_mat, embedding, W, att_i, att_j,
                              att_em_i, att_em_j)
    ssum = bmax[0, 0] + bmax[0, 1]
    B = jnp.where(ssum >= 0, ssum, 0.2 * ssum)
    acc = _sc_edges(xaug, aij.reshape(2 * N), topk_edge[0], topk_edge[1],
                    jnp.broadcast_to(B, (LANES,)))
    return _stage3(acc, xaug, aij, B.reshape(1, 1), bias, gamma, beta)
